# baseline (speedup 1.0000x reference)
import functools

import jax
import jax.numpy as jnp
from jax import lax
from jax.experimental import pallas as pl
from jax.experimental.pallas import tpu as pltpu
from jax.experimental.pallas import tpu_sc as plsc

_NBUF = 4


def _sc_gather(n, D, chunk):
    info = plsc.get_sparse_core_info()
    nw = info.num_cores * info.num_subcores
    b_per_w = n // nw
    n_chunks = b_per_w // chunk
    assert n_chunks % _NBUF == 0 and n_chunks >= 2 * _NBUF
    mesh = plsc.VectorSubcoreMesh(core_axis_name="c", subcore_axis_name="s")

    @functools.partial(
        pl.kernel,
        out_type=jax.ShapeDtypeStruct((n, D), jnp.float32),
        mesh=mesh,
        scratch_types=[
            pltpu.VMEM((b_per_w,), jnp.int32),
            [pltpu.VMEM((chunk, D), jnp.float32) for _ in range(_NBUF)],
            [pltpu.SemaphoreType.DMA for _ in range(_NBUF)],
            [pltpu.SemaphoreType.DMA for _ in range(_NBUF)],
        ],
    )
    def gather_kernel(pos_hbm, pe_hbm, out_hbm, idx_v, bufs, gsems, wsems):
        wid = lax.axis_index("s") * info.num_cores + lax.axis_index("c")
        base = wid * b_per_w
        pltpu.sync_copy(pos_hbm.at[pl.ds(base, b_per_w)], idx_v)

        def start_gather(c, b):
            pltpu.make_async_copy(
                pe_hbm.at[idx_v.at[pl.ds(c * chunk, chunk)]], bufs[b], gsems[b]
            ).start()

        def wait_gather(b):
            pltpu.make_async_copy(
                pe_hbm.at[idx_v.at[pl.ds(0, chunk)]], bufs[b], gsems[b]
            ).wait()

        def start_write(c, b):
            pltpu.make_async_copy(
                bufs[b], out_hbm.at[pl.ds(base + c * chunk, chunk)], wsems[b]
            ).start()

        def wait_write(b):
            pltpu.make_async_copy(
                bufs[b], out_hbm.at[pl.ds(base, chunk)], wsems[b]
            ).wait()

        for c in range(_NBUF - 1):
            start_gather(c, c)

        for c in range(_NBUF):
            wait_gather(c)
            start_write(c, c)
            if c > 0:
                wait_write((c + _NBUF - 1) % _NBUF)
            start_gather(c + _NBUF - 1, (c + _NBUF - 1) % _NBUF)

        @pl.loop(_NBUF, n_chunks, step=_NBUF)
        def _quad(c0):
            for k in range(_NBUF):
                c = c0 + k
                b = k
                wait_gather(b)
                start_write(c, b)

                @pl.when(c + _NBUF - 1 < n_chunks)
                def _():
                    wait_write((b + _NBUF - 1) % _NBUF)
                    start_gather(c + _NBUF - 1, (b + _NBUF - 1) % _NBUF)

        for b in range(_NBUF):
            wait_write(b)

    return gather_kernel


def kernel(positions, pe):
    B, S = positions.shape
    V, D = pe.shape
    n = B * S
    out = _sc_gather(n, D, chunk=8)(positions.reshape(n), pe)
    return out.reshape(B, S, D)
